# baseline (speedup 1.0000x reference)
import functools

import jax
import jax.numpy as jnp
from jax import lax
from jax.experimental import pallas as pl
from jax.experimental.pallas import tpu as pltpu
from jax.experimental.pallas import tpu_sc as plsc

N_NODES = 10000
N_NET = 20000
N_EDGES = 320000
N_CONN = 80000
D = 128
H1 = 64

NC = 2
NS = 16
NW = NC * NS
L = 16

DA = D + 16
EPT = 10240
E_PAD = EPT * NW
KCH = EPT // 128
ACC_ROWS = 10240
RPT = ACC_ROWS // NS

NPT = N_NET // NW
ACCC_ROWS = NPT * (D // L)
CCH = 2000
NCCH = N_CONN // CCH


def _zero_vmem_rows(ref, nrows, ncols16):
    z = jnp.zeros((L,), jnp.float32)

    def body(i, c):
        for r in range(ncols16):
            ref[i, pl.ds(r * L, L)] = z
        return c

    lax.fori_loop(0, nrows, body, 0)


def _seg_sum_body(xa_hbm, src_hbm, dst_hbm, out_hbm, src_v, dst_v, rows_v,
                  acc_sp, sem):
    cid = lax.axis_index("c")
    sid = lax.axis_index("s")

    _zero_vmem_rows(rows_v, 128, DA // L)
    base = sid * RPT
    for k in range(RPT // 128):
        pltpu.sync_copy(rows_v, acc_sp.at[pl.ds(base + k * 128, 128)])

    wid = sid * NC + cid
    pltpu.sync_copy(src_hbm.at[wid], src_v)
    pltpu.sync_copy(dst_hbm.at[wid], dst_v)
    plsc.subcore_barrier()

    def chunk(j, c):
        pltpu.async_copy(xa_hbm.at[src_v.at[j]], rows_v, sem).wait()
        pltpu.sync_copy(rows_v, acc_sp.at[dst_v.at[j]], add=True)
        return c

    lax.fori_loop(0, KCH, chunk, 0)
    plsc.subcore_barrier()

    for k in range(RPT // 128):
        pltpu.sync_copy(acc_sp.at[pl.ds(base + k * 128, 128)], rows_v)
        pltpu.sync_copy(rows_v, out_hbm.at[cid, pl.ds(base + k * 128, 128)])


def _seg_sum(xa, src3, dst3):
    mesh = plsc.VectorSubcoreMesh(core_axis_name="c", subcore_axis_name="s")
    f = functools.partial(
        pl.kernel,
        mesh=mesh,
        out_type=jax.ShapeDtypeStruct((NC, ACC_ROWS, DA), jnp.float32),
        scratch_types=[
            pltpu.VMEM((KCH, 128), jnp.int32),
            pltpu.VMEM((KCH, 128), jnp.int32),
            pltpu.VMEM((128, DA), jnp.float32),
            pltpu.VMEM_SHARED((ACC_ROWS, DA), jnp.float32),
            pltpu.SemaphoreType.DMA,
        ],
        compiler_params=pltpu.CompilerParams(use_tc_tiling_on_sc=False),
    )(_seg_sum_body)
    return f(xa, src3, dst3)


def _seg_max_body(h_hbm, csrc_hbm, cdst_hbm, y_hbm, src_ch, dst_ch, cmp_src,
                  cmp_dst, rows_v, acc, sem):
    cid = lax.axis_index("c")
    sid = lax.axis_index("s")
    wid = sid * NC + cid
    lo = wid * NPT
    hi = lo + NPT

    ninf = jnp.full((L,), -jnp.inf, jnp.float32)

    def init(i, c):
        acc[i] = ninf
        return c

    lax.fori_loop(0, ACCC_ROWS + D // L, init, 0)

    jsrc = jnp.full((L,), N_NODES, jnp.int32)
    jdst = jnp.full((L,), NPT, jnp.int32)

    def initc(i, c):
        for r in range(128 // L):
            cmp_src[i, pl.ds(r * L, L)] = jsrc
            cmp_dst[i, pl.ds(r * L, L)] = jdst
        return c

    lax.fori_loop(0, 16, initc, 0)

    def chunk(cc, c0):
        pltpu.sync_copy(csrc_hbm.at[cc], src_ch)
        pltpu.sync_copy(cdst_hbm.at[cc], dst_ch)

        def scan(i, cnt):
            dstv = dst_ch[pl.ds(i * L, L)]
            srcv = src_ch[pl.ds(i * L, L)]
            m = (dstv >= lo) & (dstv < hi)
            mi = jnp.where(m, jnp.full((L,), 1, jnp.int32),
                           jnp.full((L,), 0, jnp.int32))
            csum = plsc.cumsum(mi)
            pos = cnt + csum - 1
            plsc.store_scatter(cmp_src, [pos >> 7, pos & 127], srcv, mask=m)
            plsc.store_scatter(cmp_dst, [pos >> 7, pos & 127], dstv - lo,
                               mask=m)
            return cnt + csum[L - 1]

        cnt = lax.fori_loop(0, CCH // L, scan, 0)

        for b in range(CCH // 128 + 1):

            @pl.when(b * 128 < cnt)
            def _batch():
                pltpu.async_copy(h_hbm.at[cmp_src.at[b]], rows_v, sem).wait()

                def group(g, cg):
                    e0 = b * 128 + g * L

                    @pl.when(e0 < cnt)
                    def _grp():
                        dlv = cmp_dst[b, pl.ds(g * L, L)] * (D // L)
                        for l in range(L):
                            e = g * L + l
                            base = dlv[l]
                            for r in range(D // L):
                                acc[base + r] = jnp.maximum(
                                    acc[base + r],
                                    rows_v[e, pl.ds(r * L, L)])

                    return cg

                lax.fori_loop(0, 8, group, 0)

        return c0

    lax.fori_loop(0, NCCH, chunk, 0)

    zf = jnp.zeros((L,), jnp.float32)

    def fin(i, c):
        v = acc[i]
        acc[i] = jnp.where(v == -jnp.inf, zf, v)
        return c

    lax.fori_loop(0, ACCC_ROWS, fin, 0)
    pltpu.sync_copy(acc.at[pl.ds(0, ACCC_ROWS)], y_hbm.at[wid])


def _seg_max(h, csrc, cdst):
    mesh = plsc.VectorSubcoreMesh(core_axis_name="c", subcore_axis_name="s")
    f = functools.partial(
        pl.kernel,
        mesh=mesh,
        out_type=jax.ShapeDtypeStruct((NW, ACCC_ROWS, L), jnp.float32),
        scratch_types=[
            pltpu.VMEM((CCH,), jnp.int32),
            pltpu.VMEM((CCH,), jnp.int32),
            pltpu.VMEM((16, 128), jnp.int32),
            pltpu.VMEM((16, 128), jnp.int32),
            pltpu.VMEM((128, D), jnp.float32),
            pltpu.VMEM((ACCC_ROWS + D // L, L), jnp.float32),
            pltpu.SemaphoreType.DMA,
        ],
        compiler_params=pltpu.CompilerParams(
            use_tc_tiling_on_sc=False, needs_layout_passes=False),
    )(_seg_max_body)
    return f(h, csrc, cdst)


def _sage_body(pa_ref, pd_ref, x_ref, ws_ref, wn_ref, b_ref, h_ref):
    agg = pa_ref[0] + pa_ref[1]
    deg = pd_ref[0] + pd_ref[1]
    hn = agg / jnp.clip(deg, 1.0, None)
    h = (jnp.dot(x_ref[...], ws_ref[...], preferred_element_type=jnp.float32)
         + jnp.dot(hn, wn_ref[...], preferred_element_type=jnp.float32)
         + b_ref[...])
    h_ref[...] = jnp.where(h >= 0, h, 0.01 * h)


def _sage(pa, pd, x, ws, wn, b):
    br = 2000
    grid = (N_NODES // br,)
    return pl.pallas_call(
        _sage_body,
        out_shape=jax.ShapeDtypeStruct((N_NODES, D), jnp.float32),
        grid=grid,
        in_specs=[
            pl.BlockSpec((NC, br, D), lambda i: (0, i, 0)),
            pl.BlockSpec((NC, br, 1), lambda i: (0, i, 0)),
            pl.BlockSpec((br, D), lambda i: (i, 0)),
            pl.BlockSpec((D, D), lambda i: (0, 0)),
            pl.BlockSpec((D, D), lambda i: (0, 0)),
            pl.BlockSpec((1, D), lambda i: (0, 0)),
        ],
        out_specs=pl.BlockSpec((br, D), lambda i: (i, 0)),
    )(pa, pd, x, ws, wn, b)


def _mlp_body(y_ref, w1_ref, b1_ref, w2_ref, b2_ref, o_ref):
    xx = (jnp.dot(y_ref[...], w1_ref[...], preferred_element_type=jnp.float32)
          + b1_ref[...])
    xx = jnp.where(xx >= 0, xx, 0.01 * xx)
    o = jnp.dot(xx, w2_ref[...], preferred_element_type=jnp.float32) + b2_ref[...]
    o_ref[...] = jnp.tanh(o)


def _mlp(y, w1, b1, w2, b2):
    br = 2000
    grid = (N_NET // br,)
    return pl.pallas_call(
        _mlp_body,
        out_shape=jax.ShapeDtypeStruct((N_NET, 1), jnp.float32),
        grid=grid,
        in_specs=[
            pl.BlockSpec((br, D), lambda i: (i, 0)),
            pl.BlockSpec((D, H1), lambda i: (0, 0)),
            pl.BlockSpec((1, H1), lambda i: (0, 0)),
            pl.BlockSpec((H1, 1), lambda i: (0, 0)),
            pl.BlockSpec((1, 1), lambda i: (0, 0)),
        ],
        out_specs=pl.BlockSpec((br, 1), lambda i: (i, 0)),
    )(y, w1, b1, w2, b2)


def kernel(X, W_self, W_neigh, b_sage, W1, b1, W2, b2, edge_index, conn_src,
           conn_dst):
    xa = jnp.concatenate(
        [X, jnp.ones((N_NODES, DA - D), jnp.float32)], axis=1)
    npad = E_PAD - N_EDGES
    src3 = jnp.pad(edge_index[0], (0, npad)).reshape(NW, KCH, 128)
    dst3 = jnp.pad(edge_index[1], (0, npad),
                   constant_values=N_NODES).reshape(NW, KCH, 128)

    p = _seg_sum(xa, src3, dst3)

    pa = p[:, :N_NODES, :D]
    pd = p[:, :N_NODES, D:D + 1]
    h = _sage(pa, pd, X, W_self, W_neigh, b_sage.reshape(1, D))

    hp = jnp.concatenate([h, jnp.full((8, D), -jnp.inf, jnp.float32)], axis=0)
    y3 = _seg_max(hp, conn_src.reshape(NCCH, CCH), conn_dst.reshape(NCCH, CCH))
    y = y3.reshape(N_NET, D)

    return _mlp(y, W1, b1.reshape(1, H1), W2, b2.reshape(1, 1))


# speedup vs baseline: 1.0186x; 1.0186x over previous
import functools

import jax
import jax.numpy as jnp
from jax import lax
from jax.experimental import pallas as pl
from jax.experimental.pallas import tpu as pltpu
from jax.experimental.pallas import tpu_sc as plsc

N_NODES = 10000
N_NET = 20000
N_EDGES = 320000
N_CONN = 80000
D = 128
H1 = 64

NC = 2
NS = 16
NW = NC * NS
L = 16

DA = D + 16
EPT = 10240
E_PAD = EPT * NW
ACH = 64
KCH = EPT // ACH
ACC_ROWS = 10240
RPT = ACC_ROWS // NS

NPT = N_NET // NW
ACCC_ROWS = NPT * (D // L)
CCH = 2000
NCCH = N_CONN // CCH


def _zero_vmem_rows(ref, nrows, ncols16):
    z = jnp.zeros((L,), jnp.float32)

    def body(i, c):
        for r in range(ncols16):
            ref[i, pl.ds(r * L, L)] = z
        return c

    lax.fori_loop(0, nrows, body, 0)


def _seg_sum_body(xa_hbm, src_hbm, dst_hbm, out_hbm, src_v, dst_v, rows0,
                  rows1, acc_sp, gsem0, gsem1, ssem0, ssem1):
    cid = lax.axis_index("c")
    sid = lax.axis_index("s")

    _zero_vmem_rows(rows0, ACH, DA // L)
    base = sid * RPT
    for k in range(RPT // ACH):
        pltpu.sync_copy(rows0, acc_sp.at[pl.ds(base + k * ACH, ACH)])

    wid = sid * NC + cid
    pltpu.sync_copy(src_hbm.at[wid], src_v)
    pltpu.sync_copy(dst_hbm.at[wid], dst_v)
    plsc.subcore_barrier()

    pltpu.async_copy(xa_hbm.at[src_v.at[0]], rows0, gsem0)

    def body(i, c):
        j0 = 2 * i
        j1 = j0 + 1
        pltpu.make_async_copy(xa_hbm.at[src_v.at[j0]], rows0, gsem0).wait()

        @pl.when(i > 0)
        def _():
            pltpu.make_async_copy(
                rows1, acc_sp.at[dst_v.at[j0 - 1]], ssem1).wait()

        pltpu.async_copy(xa_hbm.at[src_v.at[j1]], rows1, gsem1)
        pltpu.async_copy(rows0, acc_sp.at[dst_v.at[j0]], ssem0, add=True)
        pltpu.make_async_copy(xa_hbm.at[src_v.at[j1]], rows1, gsem1).wait()
        pltpu.make_async_copy(rows0, acc_sp.at[dst_v.at[j0]], ssem0).wait()

        @pl.when(i < KCH // 2 - 1)
        def _():
            pltpu.async_copy(xa_hbm.at[src_v.at[j0 + 2]], rows0, gsem0)

        pltpu.async_copy(rows1, acc_sp.at[dst_v.at[j1]], ssem1, add=True)
        return c

    lax.fori_loop(0, KCH // 2, body, 0)
    pltpu.make_async_copy(rows1, acc_sp.at[dst_v.at[KCH - 1]], ssem1).wait()
    plsc.subcore_barrier()

    for k in range(RPT // ACH):
        pltpu.sync_copy(acc_sp.at[pl.ds(base + k * ACH, ACH)], rows0)
        pltpu.sync_copy(rows0, out_hbm.at[cid, pl.ds(base + k * ACH, ACH)])


def _seg_sum(xa, src3, dst3):
    mesh = plsc.VectorSubcoreMesh(core_axis_name="c", subcore_axis_name="s")
    f = functools.partial(
        pl.kernel,
        mesh=mesh,
        out_type=jax.ShapeDtypeStruct((NC, ACC_ROWS, DA), jnp.float32),
        scratch_types=[
            pltpu.VMEM((KCH, ACH), jnp.int32),
            pltpu.VMEM((KCH, ACH), jnp.int32),
            pltpu.VMEM((ACH, DA), jnp.float32),
            pltpu.VMEM((ACH, DA), jnp.float32),
            pltpu.VMEM_SHARED((ACC_ROWS, DA), jnp.float32),
            pltpu.SemaphoreType.DMA,
            pltpu.SemaphoreType.DMA,
            pltpu.SemaphoreType.DMA,
            pltpu.SemaphoreType.DMA,
        ],
        compiler_params=pltpu.CompilerParams(
            use_tc_tiling_on_sc=False, needs_layout_passes=False),
    )(_seg_sum_body)
    return f(xa, src3, dst3)


def _seg_max_body(h_hbm, csrc_hbm, cdst_hbm, y_hbm, src_a, dst_a, src_b,
                  dst_b, cmp_src, cmp_dst, rows_a, rows_b, acc, isa, ida,
                  isb, idb, ga, gb):
    cid = lax.axis_index("c")
    sid = lax.axis_index("s")
    wid = sid * NC + cid
    lo = wid * NPT
    hi = lo + NPT

    ninf = jnp.full((L,), -jnp.inf, jnp.float32)

    def init(i, c):
        for u in range(8):
            acc[i * 8 + u] = ninf
        return c

    lax.fori_loop(0, (ACCC_ROWS + D // L) // 8, init, 0)

    jsrc = jnp.full((L,), N_NODES, jnp.int32)
    jdst = jnp.full((L,), NPT, jnp.int32)

    def initc(i, c):
        cmp_src[pl.ds(i * L, L)] = jsrc
        cmp_dst[pl.ds(i * L, L)] = jdst
        return c

    lax.fori_loop(0, 2048 // L, initc, 0)

    NB = CCH // 128 + 1

    one_v = jnp.full((L,), 1, jnp.int32)
    zero_v = jnp.full((L,), 0, jnp.int32)

    def process(src_ch, dst_ch):
        def scan(i, cnt_v):
            dstv = dst_ch[pl.ds(i * L, L)]
            srcv = src_ch[pl.ds(i * L, L)]
            m = (dstv >= lo) & (dstv < hi)
            mi = jnp.where(m, one_v, zero_v)
            csum = plsc.cumsum(mi)
            pos = cnt_v + csum - mi
            plsc.store_scatter(cmp_src, [pos], srcv, mask=m)
            plsc.store_scatter(cmp_dst, [pos], dstv - lo, mask=m)
            return cnt_v + plsc.all_reduce_population_count(m)

        cnt_v = lax.fori_loop(0, CCH // L, scan, zero_v)
        cnt = cnt_v[0]

        def update(rv, b):
            def group(g, cg):
                e0 = b * 128 + g * L

                @pl.when(e0 < cnt)
                def _grp():
                    dlv = cmp_dst[pl.ds(b * 128 + g * L, L)] * (D // L)
                    for l in range(L):
                        base = dlv[l]
                        for r in range(D // L):
                            acc[base + r] = jnp.maximum(
                                acc[base + r],
                                rv[g * L + l, pl.ds(r * L, L)])

                return cg

            lax.fori_loop(0, 8, group, 0)

        @pl.when(cnt > 0)
        def _():
            pltpu.async_copy(
                h_hbm.at[cmp_src.at[pl.ds(0, 128)]], rows_a, ga)

        def pair(p, c):
            b0 = 2 * p
            b1 = b0 + 1

            @pl.when(b0 * 128 < cnt)
            def _():
                pltpu.make_async_copy(
                    h_hbm.at[cmp_src.at[pl.ds(b0 * 128, 128)]], rows_a,
                    ga).wait()

                @pl.when(b1 * 128 < cnt)
                def _():
                    pltpu.async_copy(
                        h_hbm.at[cmp_src.at[pl.ds(b1 * 128, 128)]], rows_b,
                        gb)

                update(rows_a, b0)

            @pl.when(b1 * 128 < cnt)
            def _():
                pltpu.make_async_copy(
                    h_hbm.at[cmp_src.at[pl.ds(b1 * 128, 128)]], rows_b,
                    gb).wait()

                @pl.when((b1 + 1) * 128 < cnt)
                def _():
                    pltpu.async_copy(
                        h_hbm.at[cmp_src.at[pl.ds((b1 + 1) * 128, 128)]],
                        rows_a, ga)

                update(rows_b, b1)

            return c

        lax.fori_loop(0, NB // 2, pair, 0)

    pltpu.async_copy(csrc_hbm.at[0], src_a, isa)
    pltpu.async_copy(cdst_hbm.at[0], dst_a, ida)

    def chunk2(k, c0):
        cc0 = 2 * k
        pltpu.make_async_copy(csrc_hbm.at[cc0], src_a, isa).wait()
        pltpu.make_async_copy(cdst_hbm.at[cc0], dst_a, ida).wait()
        pltpu.async_copy(csrc_hbm.at[cc0 + 1], src_b, isb)
        pltpu.async_copy(cdst_hbm.at[cc0 + 1], dst_b, idb)
        process(src_a, dst_a)
        pltpu.make_async_copy(csrc_hbm.at[cc0 + 1], src_b, isb).wait()
        pltpu.make_async_copy(cdst_hbm.at[cc0 + 1], dst_b, idb).wait()

        @pl.when(k < NCCH // 2 - 1)
        def _():
            pltpu.async_copy(csrc_hbm.at[cc0 + 2], src_a, isa)
            pltpu.async_copy(cdst_hbm.at[cc0 + 2], dst_a, ida)

        process(src_b, dst_b)
        return c0

    lax.fori_loop(0, NCCH // 2, chunk2, 0)

    zf = jnp.zeros((L,), jnp.float32)

    def fin(i, c):
        for u in range(8):
            v = acc[i * 8 + u]
            acc[i * 8 + u] = jnp.where(v == -jnp.inf, zf, v)
        return c

    lax.fori_loop(0, ACCC_ROWS // 8, fin, 0)
    pltpu.sync_copy(acc.at[pl.ds(0, ACCC_ROWS)], y_hbm.at[wid])


def _seg_max(h, csrc, cdst):
    mesh = plsc.VectorSubcoreMesh(core_axis_name="c", subcore_axis_name="s")
    f = functools.partial(
        pl.kernel,
        mesh=mesh,
        out_type=jax.ShapeDtypeStruct((NW, ACCC_ROWS, L), jnp.float32),
        scratch_types=[
            pltpu.VMEM((CCH,), jnp.int32),
            pltpu.VMEM((CCH,), jnp.int32),
            pltpu.VMEM((CCH,), jnp.int32),
            pltpu.VMEM((CCH,), jnp.int32),
            pltpu.VMEM((2048,), jnp.int32),
            pltpu.VMEM((2048,), jnp.int32),
            pltpu.VMEM((128, D), jnp.float32),
            pltpu.VMEM((128, D), jnp.float32),
            pltpu.VMEM((ACCC_ROWS + D // L, L), jnp.float32),
            pltpu.SemaphoreType.DMA,
            pltpu.SemaphoreType.DMA,
            pltpu.SemaphoreType.DMA,
            pltpu.SemaphoreType.DMA,
            pltpu.SemaphoreType.DMA,
            pltpu.SemaphoreType.DMA,
        ],
        compiler_params=pltpu.CompilerParams(
            use_tc_tiling_on_sc=False, needs_layout_passes=False),
    )(_seg_max_body)
    return f(h, csrc, cdst)


def _sage_body(pa_ref, pd_ref, x_ref, ws_ref, wn_ref, b_ref, h_ref):
    agg = pa_ref[0] + pa_ref[1]
    deg = pd_ref[0] + pd_ref[1]
    hn = agg / jnp.clip(deg, 1.0, None)
    h = (jnp.dot(x_ref[...], ws_ref[...], preferred_element_type=jnp.float32)
         + jnp.dot(hn, wn_ref[...], preferred_element_type=jnp.float32)
         + b_ref[...])
    h_ref[...] = jnp.where(h >= 0, h, 0.01 * h)


def _sage(pa, pd, x, ws, wn, b):
    br = 2000
    grid = (N_NODES // br,)
    return pl.pallas_call(
        _sage_body,
        out_shape=jax.ShapeDtypeStruct((N_NODES, D), jnp.float32),
        grid=grid,
        in_specs=[
            pl.BlockSpec((NC, br, D), lambda i: (0, i, 0)),
            pl.BlockSpec((NC, br, 1), lambda i: (0, i, 0)),
            pl.BlockSpec((br, D), lambda i: (i, 0)),
            pl.BlockSpec((D, D), lambda i: (0, 0)),
            pl.BlockSpec((D, D), lambda i: (0, 0)),
            pl.BlockSpec((1, D), lambda i: (0, 0)),
        ],
        out_specs=pl.BlockSpec((br, D), lambda i: (i, 0)),
    )(pa, pd, x, ws, wn, b)


def _mlp_body(y_ref, w1_ref, b1_ref, w2_ref, b2_ref, o_ref):
    xx = (jnp.dot(y_ref[...], w1_ref[...], preferred_element_type=jnp.float32)
          + b1_ref[...])
    xx = jnp.where(xx >= 0, xx, 0.01 * xx)
    o = jnp.dot(xx, w2_ref[...], preferred_element_type=jnp.float32) + b2_ref[...]
    o_ref[...] = jnp.tanh(o)


def _mlp(y, w1, b1, w2, b2):
    br = 2000
    grid = (N_NET // br,)
    return pl.pallas_call(
        _mlp_body,
        out_shape=jax.ShapeDtypeStruct((N_NET, 1), jnp.float32),
        grid=grid,
        in_specs=[
            pl.BlockSpec((br, D), lambda i: (i, 0)),
            pl.BlockSpec((D, H1), lambda i: (0, 0)),
            pl.BlockSpec((1, H1), lambda i: (0, 0)),
            pl.BlockSpec((H1, 1), lambda i: (0, 0)),
            pl.BlockSpec((1, 1), lambda i: (0, 0)),
        ],
        out_specs=pl.BlockSpec((br, 1), lambda i: (i, 0)),
    )(y, w1, b1, w2, b2)


def kernel(X, W_self, W_neigh, b_sage, W1, b1, W2, b2, edge_index, conn_src,
           conn_dst):
    xa = jnp.concatenate(
        [X, jnp.ones((N_NODES, DA - D), jnp.float32)], axis=1)
    npad = E_PAD - N_EDGES
    src3 = jnp.pad(edge_index[0], (0, npad)).reshape(NW, KCH, ACH)
    dst3 = jnp.pad(edge_index[1], (0, npad),
                   constant_values=N_NODES).reshape(NW, KCH, ACH)

    p = _seg_sum(xa, src3, dst3)

    pa = p[:, :N_NODES, :D]
    pd = p[:, :N_NODES, D:D + 1]
    h = _sage(pa, pd, X, W_self, W_neigh, b_sage.reshape(1, D))

    hp = jnp.concatenate([h, jnp.full((8, D), -jnp.inf, jnp.float32)], axis=0)
    y3 = _seg_max(hp, conn_src.reshape(NCCH, CCH), conn_dst.reshape(NCCH, CCH))
    y = y3.reshape(N_NET, D)

    return _mlp(y, W1, b1.reshape(1, H1), W2, b2.reshape(1, 1))
